# baseline (speedup 1.0000x reference)
import jax
import jax.numpy as jnp
from jax import lax
from jax.experimental import pallas as pl
from jax.experimental.pallas import tpu as pltpu

_CB = 8
_NBUF = 8


def _body(dpt_ref, hpt_ref, wpt_ref, out_hbm, vbuf, sems):
    ncs = pl.num_programs(0)
    pc = pl.program_id(0)
    slot = lax.rem(pc, _NBUF)
    nb = out_hbm.shape[0]
    cb, d = dpt_ref.shape
    hw = out_hbm.shape[3]

    @pl.when(pc >= _NBUF)
    def _():
        pltpu.make_async_copy(
            vbuf.at[slot], out_hbm.at[:, pl.ds(0, _CB)], sems.at[slot]
        ).wait()

    h = hpt_ref[pl.ds(pc * _CB, _CB), :]
    w = wpt_ref[pl.ds(pc * _CB, _CB), :]
    hwsum = (h[:, :, None] + w[:, None, :]).reshape(_CB, hw)
    dv = dpt_ref[pl.ds(pc * _CB, _CB), :]
    for i in range(d):
        row = hwsum + dv[:, i][:, None]
        for b in range(nb):
            vbuf[slot, b, :, i, :] = row

    pltpu.make_async_copy(
        vbuf.at[slot], out_hbm.at[:, pl.ds(pc * _CB, _CB)], sems.at[slot]
    ).start()

    @pl.when(pc == ncs - 1)
    def _():
        for s in range(_NBUF):
            pltpu.make_async_copy(
                vbuf.at[s], out_hbm.at[:, pl.ds(0, _CB)], sems.at[s]
            ).wait()


def kernel(x, d_weight, h_weight, w_weight):
    B = x.shape[0]
    d, h, w = x.shape[-3:]
    c = d_weight.shape[1]
    C = 256

    f32 = jnp.float32
    dpt = jnp.zeros((C, d), f32).at[0:c, :].set(d_weight[:d].T.astype(f32))
    hpt = jnp.zeros((C, h), f32).at[c:2 * c, :].set(h_weight[:h].T.astype(f32))
    wpt = jnp.zeros((C, w), f32).at[2 * c:C, :].set(
        w_weight[:w, : C - 2 * c].T.astype(f32))

    grid = (C // _CB,)
    out4 = pl.pallas_call(
        _body,
        grid=grid,
        in_specs=[
            pl.BlockSpec((C, d), lambda pc: (0, 0)),
            pl.BlockSpec((C, h), lambda pc: (0, 0)),
            pl.BlockSpec((C, w), lambda pc: (0, 0)),
        ],
        out_specs=pl.BlockSpec(memory_space=pltpu.HBM),
        out_shape=jax.ShapeDtypeStruct((B, C, d, h * w), f32),
        scratch_shapes=[
            pltpu.VMEM((_NBUF, B, _CB, d, h * w), f32),
            pltpu.SemaphoreType.DMA((_NBUF,)),
        ],
    )(dpt, hpt, wpt)
    return out4.reshape(B, C, d, h, w)


# speedup vs baseline: 1.0445x; 1.0445x over previous
import jax
import jax.numpy as jnp
from jax import lax
from jax.experimental import pallas as pl
from jax.experimental.pallas import tpu as pltpu

_CB = 16
_NBUF = 4


def _body(dpt_ref, hpt_ref, wpt_ref, out_hbm, vbuf, sems):
    ncs = pl.num_programs(0)
    pc = pl.program_id(0)
    slot = lax.rem(pc, _NBUF)
    nb = out_hbm.shape[0]
    cb, d = dpt_ref.shape
    hw = out_hbm.shape[3]

    @pl.when(pc >= _NBUF)
    def _():
        pltpu.make_async_copy(
            vbuf.at[slot], out_hbm.at[:, pl.ds(0, _CB)], sems.at[slot]
        ).wait()

    h = hpt_ref[pl.ds(pc * _CB, _CB), :]
    w = wpt_ref[pl.ds(pc * _CB, _CB), :]
    hwsum = (h[:, :, None] + w[:, None, :]).reshape(_CB, hw)
    dv = dpt_ref[pl.ds(pc * _CB, _CB), :]
    for i in range(d):
        row = hwsum + dv[:, i][:, None]
        for b in range(nb):
            vbuf[slot, b, :, i, :] = row

    for b in range(nb):
        pltpu.make_async_copy(
            vbuf.at[slot, b], out_hbm.at[b, pl.ds(pc * _CB, _CB)],
            sems.at[slot]
        ).start()

    @pl.when(pc == ncs - 1)
    def _():
        for s in range(_NBUF):
            pltpu.make_async_copy(
                vbuf.at[s], out_hbm.at[:, pl.ds(0, _CB)], sems.at[s]
            ).wait()


def kernel(x, d_weight, h_weight, w_weight):
    B = x.shape[0]
    d, h, w = x.shape[-3:]
    c = d_weight.shape[1]
    C = 256

    f32 = jnp.float32
    dpt = jnp.zeros((C, d), f32).at[0:c, :].set(d_weight[:d].T.astype(f32))
    hpt = jnp.zeros((C, h), f32).at[c:2 * c, :].set(h_weight[:h].T.astype(f32))
    wpt = jnp.zeros((C, w), f32).at[2 * c:C, :].set(
        w_weight[:w, : C - 2 * c].T.astype(f32))

    grid = (C // _CB,)
    out4 = pl.pallas_call(
        _body,
        grid=grid,
        in_specs=[
            pl.BlockSpec((C, d), lambda pc: (0, 0)),
            pl.BlockSpec((C, h), lambda pc: (0, 0)),
            pl.BlockSpec((C, w), lambda pc: (0, 0)),
        ],
        out_specs=pl.BlockSpec(memory_space=pltpu.HBM),
        out_shape=jax.ShapeDtypeStruct((B, C, d, h * w), f32),
        scratch_shapes=[
            pltpu.VMEM((_NBUF, B, _CB, d, h * w), f32),
            pltpu.SemaphoreType.DMA((_NBUF,)),
        ],
    )(dpt, hpt, wpt)
    return out4.reshape(B, C, d, h, w)
